# baseline (speedup 1.0000x reference)
import math
import jax
import jax.numpy as jnp
from jax.experimental import pallas as pl

G = 1000
LW = 1.0
SIZE = LW / G
SCATTER_COEF_A = 0.0136


def kernel(x, y, theta, theta_x, theta_y, p, z1a, z2a, ua, z1b, z2b, ub, nx, ny, resolution, efficiency, rad_length):
    dz = SIZE / 2.0
    mask = (x >= 0.0) & (x < LW) & (y >= 0.0) & (y < LW)
    ix = jnp.clip(jnp.floor(x / SIZE).astype(jnp.int32), 0, G - 1)
    iy = jnp.clip(jnp.floor(y / SIZE).astype(jnp.int32), 0, G - 1)
    rl = rad_length[ix, iy]
    x0 = dz / (rl * jnp.cos(theta))
    theta0 = SCATTER_COEF_A / p * jnp.sqrt(x0)
    phi_msc = ua * 2.0 * math.pi
    dh_msc = dz * jnp.sin(theta0) * (z1a / math.sqrt(12.0) + z2a / 2.0)
    dx = math.sqrt(2.0) * dh_msc * jnp.cos(phi_msc) * jnp.cos(theta_x)
    dy = math.sqrt(2.0) * dh_msc * jnp.sin(phi_msc) * jnp.cos(theta_y)
    x = jnp.where(mask, x + dx, x)
    y = jnp.where(mask, y + dy, y)
    x = x + dz * jnp.tan(theta_x)
    y = y + dz * jnp.tan(theta_y)
    mask = (x >= 0.0) & (x < LW) & (y >= 0.0) & (y < LW)
    ix = jnp.clip(jnp.floor(x / SIZE).astype(jnp.int32), 0, G - 1)
    iy = jnp.clip(jnp.floor(y / SIZE).astype(jnp.int32), 0, G - 1)
    res = jnp.where(mask, resolution[ix, iy], 0.0)
    hit_x = x + nx / (jnp.abs(res) + 1e-17)
    hit_y = y + ny / (jnp.abs(res) + 1e-17)
    return jnp.stack([hit_x, hit_y], axis=1)


# speedup vs baseline: 79.1291x; 79.1291x over previous
import functools
import math

import jax
import jax.numpy as jnp
from jax import lax
from jax.experimental import pallas as pl
from jax.experimental.pallas import tpu as pltpu
from jax.experimental.pallas import tpu_sc as plsc

_N = 2_000_000
_G = 1000
_LW = 1.0
_SIZE = _LW / _G
_DZ = _SIZE / 2.0
_A = 0.0136

_C = 8000
_NCHUNK = _N // _C
_INFO = plsc.get_sparse_core_info()
_NC = _INFO.num_cores
_NS = _INFO.num_subcores
_NW = _NC * _NS

_TB = 131072
_TGRID = (_N + _TB - 1) // _TB

_SENT = _G * _G


def _make_sc_gather(table_len):
  mesh = plsc.VectorSubcoreMesh(core_axis_name="c", subcore_axis_name="s")

  @functools.partial(
      pl.kernel, mesh=mesh,
      out_type=jax.ShapeDtypeStruct((_N,), jnp.float32),
      scratch_types=[
          pltpu.VMEM((_C,), jnp.int32),
          pltpu.VMEM((_C,), jnp.float32),
          pltpu.SemaphoreType.DMA,
      ],
  )
  def k(idx_hbm, tab_hbm, out_hbm, idxb, gatb, sem):
    wid = lax.axis_index("s") * _NC + lax.axis_index("c")

    def body(i, carry):
      cid = wid + i * _NW
      base = cid * _C
      pltpu.sync_copy(idx_hbm.at[pl.ds(base, _C)], idxb)
      pltpu.async_copy(tab_hbm.at[idxb], gatb, sem).wait()
      pltpu.sync_copy(gatb, out_hbm.at[pl.ds(base, _C)])
      return carry

    n_w = (_NCHUNK - wid + _NW - 1) // _NW
    lax.fori_loop(0, n_w, body, 0)

  return k


def _tc_idx_body(x_ref, y_ref, o_ref):
  x = x_ref[...]
  y = y_ref[...]
  ix = jnp.clip(jnp.floor(x / _SIZE).astype(jnp.int32), 0, _G - 1)
  iy = jnp.clip(jnp.floor(y / _SIZE).astype(jnp.int32), 0, _G - 1)
  o_ref[...] = ix * _G + iy


def _tc_math_body(x_ref, y_ref, th_ref, tx_ref, ty_ref, p_ref, z1_ref, z2_ref,
                  u_ref, rl_ref, xp_ref, yp_ref, f2_ref):
  x = x_ref[...]
  y = y_ref[...]
  theta = th_ref[...]
  theta_x = tx_ref[...]
  theta_y = ty_ref[...]
  p = p_ref[...]
  z1 = z1_ref[...]
  z2 = z2_ref[...]
  u = u_ref[...]
  rl = rl_ref[...]

  mask = (x >= 0.0) & (x < _LW) & (y >= 0.0) & (y < _LW)
  x0 = _DZ / (rl * jnp.cos(theta))
  theta0 = _A / p * jnp.sqrt(x0)
  phi = u * 2.0 * math.pi
  dh = _DZ * jnp.sin(theta0) * (z1 / math.sqrt(12.0) + z2 / 2.0)
  dx = math.sqrt(2.0) * dh * jnp.cos(phi) * jnp.cos(theta_x)
  dy = math.sqrt(2.0) * dh * jnp.sin(phi) * jnp.cos(theta_y)
  xn = jnp.where(mask, x + dx, x)
  yn = jnp.where(mask, y + dy, y)
  xn = xn + _DZ * jnp.tan(theta_x)
  yn = yn + _DZ * jnp.tan(theta_y)

  mask1 = (xn >= 0.0) & (xn < _LW) & (yn >= 0.0) & (yn < _LW)
  ix = jnp.clip(jnp.floor(xn / _SIZE).astype(jnp.int32), 0, _G - 1)
  iy = jnp.clip(jnp.floor(yn / _SIZE).astype(jnp.int32), 0, _G - 1)
  f2 = jnp.where(mask1, ix * _G + iy, _SENT)

  xp_ref[...] = xn
  yp_ref[...] = yn
  f2_ref[...] = f2


def _tc_hits_body(xp_ref, yp_ref, nx_ref, ny_ref, res_ref, hx_ref, hy_ref):
  d = jnp.abs(res_ref[...]) + 1e-17
  hx_ref[...] = xp_ref[...] + nx_ref[...] / d
  hy_ref[...] = yp_ref[...] + ny_ref[...] / d


_B1 = pl.BlockSpec((_TB,), lambda i: (i,))


def _elwise(body, n_in, n_out, out_dtypes):
  return pl.pallas_call(
      body,
      grid=(_TGRID,),
      in_specs=[_B1] * n_in,
      out_specs=[_B1] * n_out if n_out > 1 else _B1,
      out_shape=([jax.ShapeDtypeStruct((_N,), d) for d in out_dtypes]
                 if n_out > 1 else jax.ShapeDtypeStruct((_N,), out_dtypes[0])),
  )


def kernel(x, y, theta, theta_x, theta_y, p, z1a, z2a, ua, z1b, z2b, ub,
           nx, ny, resolution, efficiency, rad_length):
  tab1 = rad_length.reshape(-1)
  tab2 = jnp.concatenate(
      [resolution.reshape(-1), jnp.zeros((8,), jnp.float32)])

  flat1 = _elwise(_tc_idx_body, 2, 1, [jnp.int32])(x, y)
  rl = _make_sc_gather(_G * _G)(flat1, tab1)
  xp, yp, f2 = _elwise(_tc_math_body, 10, 3,
                       [jnp.float32, jnp.float32, jnp.int32])(
      x, y, theta, theta_x, theta_y, p, z1a, z2a, ua, rl)
  res = _make_sc_gather(_G * _G + 8)(f2, tab2)
  hx, hy = _elwise(_tc_hits_body, 5, 2, [jnp.float32, jnp.float32])(
      xp, yp, nx, ny, res)
  return jnp.stack([hx, hy], axis=1)
